# baseline (speedup 1.0000x reference)
import functools

import jax
import jax.numpy as jnp
from jax import lax
from jax.experimental import pallas as pl
from jax.experimental.pallas import tpu as pltpu
from jax.experimental.pallas import tpu_sc as plsc

VOCAB = 1000000
EMBED_DIM = 32
NUM_LABELS = 10
BATCH = 16384
HIST = 50
BL = BATCH * HIST

PROJ_DIM = 16
_VPB = 8192
_SUB = _VPB // 8
_SUBSH = 10
_NBLK = -(-VOCAB // _VPB)
_PROWS = _NBLK * _SUB


def _proj_body(t_ref, w_ref, b_ref, o_ref):
    tb = t_ref[...].astype(jnp.bfloat16)
    for j in range(8):
        o_ref[:, j * PROJ_DIM : (j + 1) * PROJ_DIM] = (
            lax.dot_general(
                tb[:, j * _SUB : (j + 1) * _SUB],
                w_ref[...],
                (((0,), (0,)), ((), ())),
                preferred_element_type=jnp.float32,
            )
            + b_ref[...]
        )


def _project_table(tableT, W, b):
    wp = (
        jnp.zeros((EMBED_DIM, PROJ_DIM), jnp.float32)
        .at[:, :NUM_LABELS]
        .set(W)
        .astype(jnp.bfloat16)
    )
    bp = jnp.zeros((1, PROJ_DIM), jnp.float32).at[:, :NUM_LABELS].set(b)
    p2 = pl.pallas_call(
        _proj_body,
        grid=(_NBLK,),
        in_specs=[
            pl.BlockSpec((EMBED_DIM, _VPB), lambda i: (0, i)),
            pl.BlockSpec((EMBED_DIM, PROJ_DIM), lambda i: (0, 0)),
            pl.BlockSpec((1, PROJ_DIM), lambda i: (0, 0)),
        ],
        out_specs=pl.BlockSpec((_SUB, 128), lambda i: (i, 0)),
        out_shape=jax.ShapeDtypeStruct((_PROWS, 128), jnp.float32),
        compiler_params=pltpu.CompilerParams(fuse_transposed_lhs_in_matmul=True),
    )(tableT, wp, bp)
    return p2.reshape(_PROWS * 8, PROJ_DIM)


_NC, _NS = 2, 16
_NW = _NC * _NS
_BW = BATCH // _NW


_DEPTH = 5


_NG = HIST // _DEPTH


def _gather_kernel(idx_hbm, p_hbm, out_hbm, idx_v, rows_v, ob_v, *sems):
    gsems = sems[:_DEPTH]
    isems = sems[_DEPTH : 2 * _DEPTH]
    osems = sems[2 * _DEPTH :]
    wid = lax.axis_index("s") * _NC + lax.axis_index("c")
    b0 = wid * _BW

    lane = jnp.arange(16, dtype=jnp.int32)
    cvecs = [jnp.full((16,), c, jnp.int32) for c in range(NUM_LABELS)]

    def idx_slice(k, par):
        return idx_v.at[k, pl.ds(par * _BW, _BW)]

    def rows_slice(k):
        return rows_v.at[pl.ds(k * _BW, _BW), :]

    def out_slice(l):
        return out_hbm.at[:, pl.ds(l * BATCH + b0, _BW)]

    for k in range(_DEPTH):
        pltpu.sync_copy(idx_hbm.at[pl.ds(k * BATCH + b0, _BW)], idx_slice(k, 0))
        pltpu.async_copy(p_hbm.at[idx_slice(k, 0)], rows_slice(k), gsems[k])

    def group_body(g, carry):
        l0 = g * _DEPTH
        par = (g + 1) % 2
        for k in range(_DEPTH):
            l = l0 + k

            @pl.when(g < _NG - 1)
            def _():
                pltpu.async_copy(
                    idx_hbm.at[pl.ds((l + _DEPTH) * BATCH + b0, _BW)],
                    idx_slice(k, par),
                    isems[k],
                )

            pltpu.make_async_copy(
                p_hbm.at[idx_slice(k, 1 - par)], rows_slice(k), gsems[k]
            ).wait()

            @pl.when(g > 0)
            def _():
                pltpu.make_async_copy(
                    ob_v.at[k], out_slice(l - _DEPTH), osems[k]
                ).wait()

            def pack_body(kk, c2):
                for u in range(2):
                    row = (k * _BW + kk * 32 + u * 16) + lane
                    for c in range(NUM_LABELS):
                        ob_v[k, c, pl.ds(kk * 32 + u * 16, 16)] = (
                            plsc.load_gather(rows_v, [row, cvecs[c]])
                        )
                return c2

            lax.fori_loop(0, _BW // 32, pack_body, 0)
            pltpu.async_copy(ob_v.at[k], out_slice(l), osems[k])

            @pl.when(g < _NG - 1)
            def _():
                pltpu.make_async_copy(
                    idx_hbm.at[pl.ds((l + _DEPTH) * BATCH + b0, _BW)],
                    idx_slice(k, par),
                    isems[k],
                ).wait()
                pltpu.async_copy(
                    p_hbm.at[idx_slice(k, par)], rows_slice(k), gsems[k]
                )

        return carry

    lax.fori_loop(0, _NG, group_body, 0)

    for k in range(_DEPTH):
        pltpu.make_async_copy(
            ob_v.at[k], out_slice((_NG - 1) * _DEPTH + k), osems[k]
        ).wait()


def _gather_rows(idx_flat, p):
    mesh = plsc.VectorSubcoreMesh(core_axis_name="c", subcore_axis_name="s")
    fn = functools.partial(
        pl.kernel,
        mesh=mesh,
        out_type=jax.ShapeDtypeStruct((NUM_LABELS, BL), jnp.float32),
        scratch_types=[
            pltpu.VMEM((_DEPTH, 2 * _BW), jnp.int32),
            pltpu.VMEM((_DEPTH * _BW, PROJ_DIM), jnp.float32),
            pltpu.VMEM((_DEPTH, NUM_LABELS, _BW), jnp.float32),
        ]
        + [pltpu.SemaphoreType.DMA] * (3 * _DEPTH),
        compiler_params=pltpu.CompilerParams(
            use_tc_tiling_on_sc=False,
            needs_layout_passes=False,
            disable_bounds_checks=True,
        ),
    )(_gather_kernel)
    return fn(idx_flat, p)


def kernel(x, table, W, b):
    p = _project_table(table.T, W, b)
    v = x.T.reshape(BL)
    xp = (
        (v & ~jnp.int32(_VPB - 1))
        + ((v & (_SUB - 1)) << 3)
        + ((v >> _SUBSH) & 7)
    )
    out = _gather_rows(xp, p)
    return out.reshape(NUM_LABELS, HIST, BATCH).transpose(2, 1, 0)


# speedup vs baseline: 1.0212x; 1.0212x over previous
import functools

import jax
import jax.numpy as jnp
from jax import lax
from jax.experimental import pallas as pl
from jax.experimental.pallas import tpu as pltpu
from jax.experimental.pallas import tpu_sc as plsc

VOCAB = 1000000
EMBED_DIM = 32
NUM_LABELS = 10
BATCH = 16384
HIST = 50
BL = BATCH * HIST

PROJ_DIM = 16
_VPB = 16384
_SUB = _VPB // 8
_SUBSH = 11
_NBLK = -(-VOCAB // _VPB)
_PROWS = _NBLK * _SUB


def _proj_body(t_ref, w_ref, b_ref, o_ref):
    tb = t_ref[...].astype(jnp.bfloat16)
    for j in range(8):
        o_ref[:, j * PROJ_DIM : (j + 1) * PROJ_DIM] = (
            lax.dot_general(
                tb[:, j * _SUB : (j + 1) * _SUB],
                w_ref[...],
                (((0,), (0,)), ((), ())),
                preferred_element_type=jnp.float32,
            )
            + b_ref[...]
        )


def _project_table(tableT, W, b):
    wp = (
        jnp.zeros((EMBED_DIM, PROJ_DIM), jnp.float32)
        .at[:, :NUM_LABELS]
        .set(W)
        .astype(jnp.bfloat16)
    )
    bp = jnp.zeros((1, PROJ_DIM), jnp.float32).at[:, :NUM_LABELS].set(b)
    p2 = pl.pallas_call(
        _proj_body,
        grid=(_NBLK,),
        in_specs=[
            pl.BlockSpec((EMBED_DIM, _VPB), lambda i: (0, i)),
            pl.BlockSpec((EMBED_DIM, PROJ_DIM), lambda i: (0, 0)),
            pl.BlockSpec((1, PROJ_DIM), lambda i: (0, 0)),
        ],
        out_specs=pl.BlockSpec((_SUB, 128), lambda i: (i, 0)),
        out_shape=jax.ShapeDtypeStruct((_PROWS, 128), jnp.float32),
        compiler_params=pltpu.CompilerParams(fuse_transposed_lhs_in_matmul=True),
    )(tableT, wp, bp)
    return p2.reshape(_PROWS * 8, PROJ_DIM)


_NC, _NS = 2, 16
_NW = _NC * _NS
_BW = BATCH // _NW


_DEPTH = 5


_NG = HIST // _DEPTH


def _gather_kernel(idx_hbm, p_hbm, out_hbm, idx_v, rows_v, ob_v, *sems):
    gsems = sems[:_DEPTH]
    isems = sems[_DEPTH : 2 * _DEPTH]
    osems = sems[2 * _DEPTH :]
    wid = lax.axis_index("s") * _NC + lax.axis_index("c")
    b0 = wid * _BW

    lane = jnp.arange(16, dtype=jnp.int32)
    cvecs = [jnp.full((16,), c, jnp.int32) for c in range(NUM_LABELS)]

    def idx_slice(k, par):
        return idx_v.at[k, pl.ds(par * _BW, _BW)]

    def rows_slice(k):
        return rows_v.at[pl.ds(k * _BW, _BW), :]

    def out_slice(l):
        return out_hbm.at[:, pl.ds(l * BATCH + b0, _BW)]

    for k in range(_DEPTH):
        pltpu.sync_copy(idx_hbm.at[pl.ds(k * BATCH + b0, _BW)], idx_slice(k, 0))
        pltpu.async_copy(p_hbm.at[idx_slice(k, 0)], rows_slice(k), gsems[k])

    def group_body(g, carry):
        l0 = g * _DEPTH
        par = (g + 1) % 2
        for k in range(_DEPTH):
            l = l0 + k

            @pl.when(g < _NG - 1)
            def _():
                pltpu.async_copy(
                    idx_hbm.at[pl.ds((l + _DEPTH) * BATCH + b0, _BW)],
                    idx_slice(k, par),
                    isems[k],
                )

            pltpu.make_async_copy(
                p_hbm.at[idx_slice(k, 1 - par)], rows_slice(k), gsems[k]
            ).wait()

            @pl.when(g > 0)
            def _():
                pltpu.make_async_copy(
                    ob_v.at[k], out_slice(l - _DEPTH), osems[k]
                ).wait()

            def pack_body(kk, c2):
                for u in range(2):
                    row = (k * _BW + kk * 32 + u * 16) + lane
                    for c in range(NUM_LABELS):
                        ob_v[k, c, pl.ds(kk * 32 + u * 16, 16)] = (
                            plsc.load_gather(rows_v, [row, cvecs[c]])
                        )
                return c2

            lax.fori_loop(0, _BW // 32, pack_body, 0)
            pltpu.async_copy(ob_v.at[k], out_slice(l), osems[k])

            @pl.when(g < _NG - 1)
            def _():
                pltpu.make_async_copy(
                    idx_hbm.at[pl.ds((l + _DEPTH) * BATCH + b0, _BW)],
                    idx_slice(k, par),
                    isems[k],
                ).wait()
                pltpu.async_copy(
                    p_hbm.at[idx_slice(k, par)], rows_slice(k), gsems[k]
                )

        return carry

    lax.fori_loop(0, _NG, group_body, 0)

    for k in range(_DEPTH):
        pltpu.make_async_copy(
            ob_v.at[k], out_slice((_NG - 1) * _DEPTH + k), osems[k]
        ).wait()


def _gather_rows(idx_flat, p):
    mesh = plsc.VectorSubcoreMesh(core_axis_name="c", subcore_axis_name="s")
    fn = functools.partial(
        pl.kernel,
        mesh=mesh,
        out_type=jax.ShapeDtypeStruct((NUM_LABELS, BL), jnp.float32),
        scratch_types=[
            pltpu.VMEM((_DEPTH, 2 * _BW), jnp.int32),
            pltpu.VMEM((_DEPTH * _BW, PROJ_DIM), jnp.float32),
            pltpu.VMEM((_DEPTH, NUM_LABELS, _BW), jnp.float32),
        ]
        + [pltpu.SemaphoreType.DMA] * (3 * _DEPTH),
        compiler_params=pltpu.CompilerParams(
            use_tc_tiling_on_sc=False,
            needs_layout_passes=False,
            disable_bounds_checks=True,
        ),
    )(_gather_kernel)
    return fn(idx_flat, p)


def kernel(x, table, W, b):
    p = _project_table(table.T, W, b)
    v = x.T.reshape(BL)
    xp = (
        (v & ~jnp.int32(_VPB - 1))
        + ((v & (_SUB - 1)) << 3)
        + ((v >> _SUBSH) & 7)
    )
    out = _gather_rows(xp, p)
    return out.reshape(NUM_LABELS, HIST, BATCH).transpose(2, 1, 0)
